# baseline (speedup 1.0000x reference)
import jax
import jax.numpy as jnp
from jax.experimental import pallas as pl
from jax.experimental.pallas import tpu as pltpu

B, H, S, D, Q = 16, 16, 2048, 128, 16
BH = B * H
NCHUNK = 16
CB = BH // NCHUNK


def _body(pos_ref, kc_ref, vc_ref, kv_ref, vv_ref, ko_ref, vo_ref,
          bulk_sem, scat_sem):
    for c in range(NCHUNK):
        sl = pl.ds(c * CB, CB)
        pltpu.make_async_copy(kc_ref.at[sl], ko_ref.at[sl],
                              bulk_sem.at[2 * c]).start()
        pltpu.make_async_copy(vc_ref.at[sl], vo_ref.at[sl],
                              bulk_sem.at[2 * c + 1]).start()
    for c in range(NCHUNK):
        pltpu.make_async_copy(kc_ref.at[pl.ds(c * CB, CB)],
                              ko_ref.at[pl.ds(c * CB, CB)],
                              bulk_sem.at[2 * c]).wait()
        pltpu.make_async_copy(vc_ref.at[pl.ds(c * CB, CB)],
                              vo_ref.at[pl.ds(c * CB, CB)],
                              bulk_sem.at[2 * c + 1]).wait()
    for q in range(0):
        p = pos_ref[q]
        pltpu.make_async_copy(kv_ref.at[:, pl.ds(q, 1), :],
                              ko_ref.at[:, pl.ds(p, 1), :],
                              scat_sem.at[2 * q]).start()
        pltpu.make_async_copy(vv_ref.at[:, pl.ds(q, 1), :],
                              vo_ref.at[:, pl.ds(p, 1), :],
                              scat_sem.at[2 * q + 1]).start()
    for q in range(0):
        p = pos_ref[q]
        pltpu.make_async_copy(kv_ref.at[:, pl.ds(q, 1), :],
                              ko_ref.at[:, pl.ds(p, 1), :],
                              scat_sem.at[2 * q]).wait()
        pltpu.make_async_copy(vv_ref.at[:, pl.ds(q, 1), :],
                              vo_ref.at[:, pl.ds(p, 1), :],
                              scat_sem.at[2 * q + 1]).wait()


def kernel(k_cache, v_cache, input_pos, k_val, v_val):
    kc = k_cache.reshape(BH, S, D)
    vc = v_cache.reshape(BH, S, D)
    kv = k_val.reshape(BH, Q, D)
    vv = v_val.reshape(BH, Q, D)

    grid_spec = pltpu.PrefetchScalarGridSpec(
        num_scalar_prefetch=1,
        grid=(1,),
        in_specs=[
            pl.BlockSpec(memory_space=pltpu.MemorySpace.HBM),
            pl.BlockSpec(memory_space=pltpu.MemorySpace.HBM),
            pl.BlockSpec((BH, Q, D), lambda i, pos: (0, 0, 0)),
            pl.BlockSpec((BH, Q, D), lambda i, pos: (0, 0, 0)),
        ],
        out_specs=[
            pl.BlockSpec(memory_space=pltpu.MemorySpace.HBM),
            pl.BlockSpec(memory_space=pltpu.MemorySpace.HBM),
        ],
        scratch_shapes=[
            pltpu.SemaphoreType.DMA((2 * NCHUNK,)),
            pltpu.SemaphoreType.DMA((2 * Q,)),
        ],
    )

    k_out, v_out = pl.pallas_call(
        _body,
        grid_spec=grid_spec,
        out_shape=[
            jax.ShapeDtypeStruct((BH, S, D), jnp.float32),
            jax.ShapeDtypeStruct((BH, S, D), jnp.float32),
        ],
    )(input_pos, kc, vc, kv, vv)

    return (k_out.reshape(B, H, S, D), v_out.reshape(B, H, S, D))


# speedup vs baseline: 48.6483x; 48.6483x over previous
import jax
import jax.numpy as jnp
from jax.experimental import pallas as pl
from jax.experimental.pallas import tpu as pltpu

B, H, S, D, Q = 16, 16, 2048, 128, 16


CB = 4


def _body(pos_ref, kc_ref, vc_ref, kv_ref, vv_ref, ko_ref, vo_ref):
    ko_ref[...] = kc_ref[...]
    vo_ref[...] = vc_ref[...]
    for c in range(CB):
        for q in range(Q):
            p = pos_ref[q]
            ko_ref[c, pl.ds(p, 1), :] = kv_ref[c, pl.ds(q, 1), :]
            vo_ref[c, pl.ds(p, 1), :] = vv_ref[c, pl.ds(q, 1), :]


def kernel(k_cache, v_cache, input_pos, k_val, v_val):
    BH = B * H
    kc = k_cache.reshape(BH, S, D)
    vc = v_cache.reshape(BH, S, D)
    kv = k_val.reshape(BH, Q, D)
    vv = v_val.reshape(BH, Q, D)

    grid_spec = pltpu.PrefetchScalarGridSpec(
        num_scalar_prefetch=1,
        grid=(BH // CB,),
        in_specs=[
            pl.BlockSpec((CB, S, D), lambda i, pos: (i, 0, 0)),
            pl.BlockSpec((CB, S, D), lambda i, pos: (i, 0, 0)),
            pl.BlockSpec((CB, Q, D), lambda i, pos: (i, 0, 0)),
            pl.BlockSpec((CB, Q, D), lambda i, pos: (i, 0, 0)),
        ],
        out_specs=[
            pl.BlockSpec((CB, S, D), lambda i, pos: (i, 0, 0)),
            pl.BlockSpec((CB, S, D), lambda i, pos: (i, 0, 0)),
        ],
    )

    k_out, v_out = pl.pallas_call(
        _body,
        grid_spec=grid_spec,
        out_shape=[
            jax.ShapeDtypeStruct((BH, S, D), jnp.float32),
            jax.ShapeDtypeStruct((BH, S, D), jnp.float32),
        ],
        compiler_params=pltpu.CompilerParams(
            dimension_semantics=("arbitrary",),
        ),
    )(input_pos, kc, vc, kv, vv)

    return (k_out.reshape(B, H, S, D), v_out.reshape(B, H, S, D))
